# baseline (speedup 1.0000x reference)
import functools

import jax
import jax.numpy as jnp
from jax import lax
from jax.experimental import pallas as pl
from jax.experimental.pallas import tpu as pltpu
from jax.experimental.pallas import tpu_sc as plsc

BATCH = 16384
VOCAB = 100000
DIM = 64
NUM_CORES = 2
NUM_SUBCORES = 16
NW = NUM_CORES * NUM_SUBCORES
BPW = BATCH // NW
CHUNK = 256
VBLK = 8192
NCHUNK = BPW // CHUNK


def _combo_body(utT_ref, itT_ref, ws_ref, wto_ref, o_ref):
    s = jnp.concatenate([utT_ref[...], itT_ref[...]], axis=0)
    st = jnp.transpose(s, (1, 0))
    w = jnp.concatenate([ws_ref[...], wto_ref[...]], axis=0)
    wt = jnp.transpose(w, (1, 0))
    f = jnp.dot(st[:, 0:DIM], wt, preferred_element_type=jnp.float32)
    o_ref[...] = jnp.concatenate([f, st[:, DIM:2 * DIM]], axis=1)


def _build_combo(utT, itT, ws, wto):
    grid = (VOCAB + VBLK - 1) // VBLK
    return pl.pallas_call(
        _combo_body,
        grid=(grid,),
        in_specs=[
            pl.BlockSpec((DIM, VBLK), lambda j: (0, j)),
            pl.BlockSpec((DIM, VBLK), lambda j: (0, j)),
            pl.BlockSpec((DIM // 2, DIM), lambda j: (0, 0)),
            pl.BlockSpec((DIM // 2, DIM), lambda j: (0, 0)),
        ],
        out_specs=pl.BlockSpec((VBLK, 2 * DIM), lambda j: (j, 0)),
        out_shape=jax.ShapeDtypeStruct((VOCAB, 2 * DIM), jnp.float32),
    )(utT, itT, ws, wto)


def _gather_dot_body(combo_hbm, x0_hbm, x1_hbm, out_hbm,
                     idx_u, idx_i, rows_u0, rows_i0, rows_u1, rows_i1,
                     staged, out_v, sem_u0, sem_i0, sem_u1, sem_i1):
    wid = lax.axis_index("s") * NUM_CORES + lax.axis_index("c")
    base = wid * BPW
    pltpu.sync_copy(x0_hbm.at[pl.ds(base, BPW)], idx_u)
    pltpu.sync_copy(x1_hbm.at[pl.ds(base, BPW)], idx_i)

    lanes = lax.iota(jnp.int32, 16)
    rows_u = (rows_u0, rows_u1)
    rows_i = (rows_i0, rows_i1)
    sems_u = (sem_u0, sem_u1)
    sems_i = (sem_i0, sem_i1)
    NBUF = 2

    def fire(ci):
        b = ci % NBUF
        off = ci * CHUNK
        cu = pltpu.async_copy(combo_hbm.at[idx_u.at[pl.ds(off, CHUNK)]],
                              rows_u[b], sems_u[b])
        cx = pltpu.async_copy(combo_hbm.at[idx_i.at[pl.ds(off, CHUNK)]],
                              rows_i[b], sems_i[b])
        return cu, cx

    def consume(ci, handles):
        b = ci % NBUF
        off = ci * CHUNK
        cu, cx = handles
        cu.wait()
        cx.wait()
        ru, ri = rows_u[b], rows_i[b]

        def row_step(r, _):
            acc = ru[r, pl.ds(0, 16)] * ri[r, pl.ds(0, 16)]
            for c in range(1, DIM // 16):
                acc = acc + (ru[r, pl.ds(16 * c, 16)]
                             * ri[r, pl.ds(16 * c, 16)])
            staged[r, pl.ds(0, 16)] = acc
            return 0

        lax.fori_loop(0, CHUNK, row_step, 0, unroll=8)

        def red_step(g, _):
            row0 = g * 16
            s = plsc.load_gather(staged, [row0 + lanes,
                                          jnp.zeros((16,), jnp.int32)])
            for k in range(1, 16):
                s = s + plsc.load_gather(
                    staged, [row0 + lanes, jnp.full((16,), k, jnp.int32)])
            out_v[pl.ds(off + row0, 16)] = 1.0 / (1.0 + jnp.exp(-s))
            return 0

        lax.fori_loop(0, CHUNK // 16, red_step, 0)

    inflight = [fire(ci) for ci in range(min(NBUF, NCHUNK))]
    for ci in range(NCHUNK):
        nf = ci + NBUF
        if nf < NCHUNK:
            consume(ci, inflight[0])
            inflight = inflight[1:] + [fire(nf)]
        else:
            consume(ci, inflight[0])
            inflight = inflight[1:]

    pltpu.sync_copy(out_v, out_hbm.at[pl.ds(base, BPW)])


def _gather_dot(combo, x0, x1):
    combo = jnp.reshape(combo, (2 * VOCAB, DIM))
    k = pl.kernel(
        _gather_dot_body,
        out_type=jax.ShapeDtypeStruct((BATCH,), jnp.float32),
        mesh=plsc.VectorSubcoreMesh(core_axis_name="c", subcore_axis_name="s"),
        scratch_types=[
            pltpu.VMEM((BPW,), jnp.int32),
            pltpu.VMEM((BPW,), jnp.int32),
            pltpu.VMEM((CHUNK, DIM), jnp.float32),
            pltpu.VMEM((CHUNK, DIM), jnp.float32),
            pltpu.VMEM((CHUNK, DIM), jnp.float32),
            pltpu.VMEM((CHUNK, DIM), jnp.float32),
            pltpu.VMEM((CHUNK, 17), jnp.float32),
            pltpu.VMEM((BPW,), jnp.float32),
            pltpu.SemaphoreType.DMA,
            pltpu.SemaphoreType.DMA,
            pltpu.SemaphoreType.DMA,
            pltpu.SemaphoreType.DMA,
        ],
        compiler_params=pltpu.CompilerParams(use_tc_tiling_on_sc=False,
                                             needs_layout_passes=False),
    )
    return k(combo, x0, x1)


def kernel(x, target_user_table, target_item_table, W_shared, W_target_only):
    x0 = x[:, 0].astype(jnp.int32) * 2
    x1 = x[:, 1].astype(jnp.int32) * 2 + 1
    combo = _build_combo(target_user_table.T, target_item_table.T,
                         W_shared, W_target_only)
    return _gather_dot(combo, x0, x1)


# speedup vs baseline: 1.0062x; 1.0062x over previous
import functools

import jax
import jax.numpy as jnp
from jax import lax
from jax.experimental import pallas as pl
from jax.experimental.pallas import tpu as pltpu
from jax.experimental.pallas import tpu_sc as plsc

BATCH = 16384
VOCAB = 100000
DIM = 64
NUM_CORES = 2
NUM_SUBCORES = 16
NW = NUM_CORES * NUM_SUBCORES
BPW = BATCH // NW
CHUNK = 128
VBLK = 8192
NCHUNK = BPW // CHUNK


def _combo_body(utT_ref, itT_ref, ws_ref, wto_ref, o_ref):
    s = jnp.concatenate([utT_ref[...], itT_ref[...]], axis=0)
    st = jnp.transpose(s, (1, 0))
    w = jnp.concatenate([ws_ref[...], wto_ref[...]], axis=0)
    wt = jnp.transpose(w, (1, 0))
    f = jnp.dot(st[:, 0:DIM], wt, preferred_element_type=jnp.float32)
    o_ref[...] = jnp.concatenate([f, st[:, DIM:2 * DIM]], axis=1)


def _build_combo(utT, itT, ws, wto):
    grid = (VOCAB + VBLK - 1) // VBLK
    return pl.pallas_call(
        _combo_body,
        grid=(grid,),
        in_specs=[
            pl.BlockSpec((DIM, VBLK), lambda j: (0, j)),
            pl.BlockSpec((DIM, VBLK), lambda j: (0, j)),
            pl.BlockSpec((DIM // 2, DIM), lambda j: (0, 0)),
            pl.BlockSpec((DIM // 2, DIM), lambda j: (0, 0)),
        ],
        out_specs=pl.BlockSpec((VBLK, 2 * DIM), lambda j: (j, 0)),
        out_shape=jax.ShapeDtypeStruct((VOCAB, 2 * DIM), jnp.float32),
    )(utT, itT, ws, wto)


def _gather_dot_body(combo_hbm, x0_hbm, x1_hbm, out_hbm,
                     idx_u, idx_i, rows_u0, rows_i0, rows_u1, rows_i1,
                     rows_u2, rows_i2, staged, out_v,
                     sem_u0, sem_i0, sem_u1, sem_i1, sem_u2, sem_i2):
    wid = lax.axis_index("s") * NUM_CORES + lax.axis_index("c")
    base = wid * BPW
    pltpu.sync_copy(x0_hbm.at[pl.ds(base, BPW)], idx_u)
    pltpu.sync_copy(x1_hbm.at[pl.ds(base, BPW)], idx_i)

    lanes = lax.iota(jnp.int32, 16)
    rows_u = (rows_u0, rows_u1, rows_u2)
    rows_i = (rows_i0, rows_i1, rows_i2)
    sems_u = (sem_u0, sem_u1, sem_u2)
    sems_i = (sem_i0, sem_i1, sem_i2)
    NBUF = 3

    def fire(ci):
        b = ci % NBUF
        off = ci * CHUNK
        cu = pltpu.async_copy(combo_hbm.at[idx_u.at[pl.ds(off, CHUNK)]],
                              rows_u[b], sems_u[b])
        cx = pltpu.async_copy(combo_hbm.at[idx_i.at[pl.ds(off, CHUNK)]],
                              rows_i[b], sems_i[b])
        return cu, cx

    def consume(ci, handles):
        b = ci % NBUF
        off = ci * CHUNK
        cu, cx = handles
        cu.wait()
        cx.wait()
        ru, ri = rows_u[b], rows_i[b]

        def row_step(r, _):
            acc = ru[r, pl.ds(0, 16)] * ri[r, pl.ds(0, 16)]
            for c in range(1, DIM // 16):
                acc = acc + (ru[r, pl.ds(16 * c, 16)]
                             * ri[r, pl.ds(16 * c, 16)])
            staged[r, pl.ds(0, 16)] = acc
            return 0

        lax.fori_loop(0, CHUNK, row_step, 0, unroll=8)

        def red_step(g, _):
            row0 = g * 16
            s = plsc.load_gather(staged, [row0 + lanes,
                                          jnp.zeros((16,), jnp.int32)])
            for k in range(1, 16):
                s = s + plsc.load_gather(
                    staged, [row0 + lanes, jnp.full((16,), k, jnp.int32)])
            out_v[pl.ds(off + row0, 16)] = 1.0 / (1.0 + jnp.exp(-s))
            return 0

        lax.fori_loop(0, CHUNK // 16, red_step, 0)

    inflight = [fire(ci) for ci in range(min(NBUF, NCHUNK))]
    for ci in range(NCHUNK):
        nf = ci + NBUF
        if nf < NCHUNK:
            consume(ci, inflight[0])
            inflight = inflight[1:] + [fire(nf)]
        else:
            consume(ci, inflight[0])
            inflight = inflight[1:]

    pltpu.sync_copy(out_v, out_hbm.at[pl.ds(base, BPW)])


def _gather_dot(combo, x0, x1):
    combo = jnp.reshape(combo, (2 * VOCAB, DIM))
    k = pl.kernel(
        _gather_dot_body,
        out_type=jax.ShapeDtypeStruct((BATCH,), jnp.float32),
        mesh=plsc.VectorSubcoreMesh(core_axis_name="c", subcore_axis_name="s"),
        scratch_types=[
            pltpu.VMEM((BPW,), jnp.int32),
            pltpu.VMEM((BPW,), jnp.int32),
            pltpu.VMEM((CHUNK, DIM), jnp.float32),
            pltpu.VMEM((CHUNK, DIM), jnp.float32),
            pltpu.VMEM((CHUNK, DIM), jnp.float32),
            pltpu.VMEM((CHUNK, DIM), jnp.float32),
            pltpu.VMEM((CHUNK, DIM), jnp.float32),
            pltpu.VMEM((CHUNK, DIM), jnp.float32),
            pltpu.VMEM((CHUNK, 17), jnp.float32),
            pltpu.VMEM((BPW,), jnp.float32),
            pltpu.SemaphoreType.DMA,
            pltpu.SemaphoreType.DMA,
            pltpu.SemaphoreType.DMA,
            pltpu.SemaphoreType.DMA,
            pltpu.SemaphoreType.DMA,
            pltpu.SemaphoreType.DMA,
        ],
        compiler_params=pltpu.CompilerParams(use_tc_tiling_on_sc=False,
                                             needs_layout_passes=False),
    )
    return k(combo, x0, x1)


def kernel(x, target_user_table, target_item_table, W_shared, W_target_only):
    x0 = x[:, 0].astype(jnp.int32) * 2
    x1 = x[:, 1].astype(jnp.int32) * 2 + 1
    combo = _build_combo(target_user_table.T, target_item_table.T,
                         W_shared, W_target_only)
    return _gather_dot(combo, x0, x1)
